# baseline (speedup 1.0000x reference)
import functools

import jax
import jax.numpy as jnp
from jax import lax
from jax.experimental import pallas as pl
from jax.experimental.pallas import tpu as pltpu
from jax.experimental.pallas import tpu_sc as plsc

B, N, K, D = 4096, 200, 50, 128
NC, NS, L = 2, 16, 16
NW = NC * NS
ROWS = B * K
RPW = ROWS // NW
CHUNK = 64
NCHUNK = RPW // CHUNK
NB = 14

_mesh = plsc.VectorSubcoreMesh(
    core_axis_name="c", subcore_axis_name="s", num_cores=NC, num_subcores=NS
)


@functools.partial(
    pl.kernel,
    out_type=jax.ShapeDtypeStruct((ROWS, D), jnp.float32),
    mesh=_mesh,
    scratch_types=[
        pltpu.VMEM((RPW,), jnp.int32),
        pltpu.VMEM((NB, CHUNK, D), jnp.float32),
    ]
    + [pltpu.SemaphoreType.DMA] * (2 * NB),
    compiler_params=pltpu.CompilerParams(
        needs_layout_passes=False, use_tc_tiling_on_sc=True
    ),
)
def _gather(src_hbm, idx_hbm, out_hbm, idx_v, rows, *sems):
    sem_g = sems[:NB]
    sem_o = sems[NB:]
    wid = lax.axis_index("s") * NC + lax.axis_index("c")
    wbase = wid * RPW
    lane = lax.iota(jnp.int32, L)

    pltpu.sync_copy(idx_hbm.at[pl.ds(wbase, RPW)], idx_v)

    def flatten_chunk(g):
        for u in range(CHUNK // L):
            off = g * CHUNK + u * L
            b = lax.bitwise_and(wbase + off + lane, B - 1)
            idx_v[pl.ds(off, L)] = b * N + idx_v[pl.ds(off, L)]

    gathers = {}
    writes = {}

    def start_gather(g):
        gathers[g] = pltpu.async_copy(
            src_hbm.at[idx_v.at[pl.ds(g * CHUNK, CHUNK)]],
            rows.at[g % NB],
            sem_g[g % NB],
        )

    def start_write(g):
        writes[g] = pltpu.async_copy(
            rows.at[g % NB],
            out_hbm.at[pl.ds(wbase + g * CHUNK, CHUNK)],
            sem_o[g % NB],
        )

    for g in range(NCHUNK):
        flatten_chunk(g)
        if g >= NB:
            writes[g - NB].wait()
        start_gather(g)
        if g >= NB - 1:
            gathers[g - (NB - 1)].wait()
            start_write(g - (NB - 1))
    for g in range(NCHUNK - (NB - 1), NCHUNK):
        gathers[g].wait()
        start_write(g)
    for g in range(NCHUNK - NB, NCHUNK):
        writes[g].wait()


def kernel(source, idxs):
    src = source.reshape(B * N, D)
    idx_kmajor = idxs.astype(jnp.int32)[..., 0].T.reshape(ROWS)
    out = _gather(src, idx_kmajor)
    return out.reshape(K, B, D).transpose(1, 0, 2)


# speedup vs baseline: 1.0138x; 1.0138x over previous
import functools

import jax
import jax.numpy as jnp
from jax import lax
from jax.experimental import pallas as pl
from jax.experimental.pallas import tpu as pltpu
from jax.experimental.pallas import tpu_sc as plsc

B, N, K, D = 4096, 200, 50, 128
NC, NS, L = 2, 16, 16
NW = NC * NS
ROWS = B * K
RPW = ROWS // NW
CHUNK = 128
NCHUNK = RPW // CHUNK
NB = 7

_mesh = plsc.VectorSubcoreMesh(
    core_axis_name="c", subcore_axis_name="s", num_cores=NC, num_subcores=NS
)


@functools.partial(
    pl.kernel,
    out_type=jax.ShapeDtypeStruct((ROWS, D), jnp.float32),
    mesh=_mesh,
    scratch_types=[
        pltpu.VMEM((RPW,), jnp.int32),
        pltpu.VMEM((NB, CHUNK, D), jnp.float32),
    ]
    + [pltpu.SemaphoreType.DMA] * (2 * NB),
    compiler_params=pltpu.CompilerParams(
        needs_layout_passes=False, use_tc_tiling_on_sc=True
    ),
)
def _gather(src_hbm, idx_hbm, out_hbm, idx_v, rows, *sems):
    sem_g = sems[:NB]
    sem_o = sems[NB:]
    wid = lax.axis_index("s") * NC + lax.axis_index("c")
    wbase = wid * RPW
    lane = lax.iota(jnp.int32, L)

    pltpu.sync_copy(idx_hbm.at[pl.ds(wbase, RPW)], idx_v)

    def flatten_chunk(g):
        for u in range(CHUNK // L):
            off = g * CHUNK + u * L
            b = lax.bitwise_and(wbase + off + lane, B - 1)
            idx_v[pl.ds(off, L)] = b * N + idx_v[pl.ds(off, L)]

    gathers = {}
    writes = {}

    def start_gather(g):
        gathers[g] = pltpu.async_copy(
            src_hbm.at[idx_v.at[pl.ds(g * CHUNK, CHUNK)]],
            rows.at[g % NB],
            sem_g[g % NB],
        )

    def start_write(g):
        writes[g] = pltpu.async_copy(
            rows.at[g % NB],
            out_hbm.at[pl.ds(wbase + g * CHUNK, CHUNK)],
            sem_o[g % NB],
        )

    for g in range(NCHUNK):
        flatten_chunk(g)
        if g >= NB:
            writes[g - NB].wait()
        start_gather(g)
        if g >= NB - 1:
            gathers[g - (NB - 1)].wait()
            start_write(g - (NB - 1))
    for g in range(NCHUNK - (NB - 1), NCHUNK):
        gathers[g].wait()
        start_write(g)
    for g in range(NCHUNK - NB, NCHUNK):
        writes[g].wait()


def kernel(source, idxs):
    src = source.reshape(B * N, D)
    idx_kmajor = idxs.astype(jnp.int32)[..., 0].T.reshape(ROWS)
    out = _gather(src, idx_kmajor)
    return out.reshape(K, B, D).transpose(1, 0, 2)
